# baseline (speedup 1.0000x reference)
import functools

import jax
import jax.numpy as jnp
from jax import lax
from jax.experimental import pallas as pl
from jax.experimental.pallas import tpu as pltpu
from jax.experimental.pallas import tpu_sc as plsc

B = 16384
E = 32
NF = 64
PACK = 4
GW = PACK * E

_info = plsc.get_sparse_core_info()
_NC, _NS = _info.num_cores, _info.num_subcores
NW = _NC * _NS
BPW = B // NW
L = 16


@functools.partial(
    pl.kernel,
    mesh=plsc.VectorSubcoreMesh(core_axis_name="c", subcore_axis_name="s"),
    out_type=[
        jax.ShapeDtypeStruct((B, GW), jnp.float32),
        jax.ShapeDtypeStruct((B, GW), jnp.float32),
    ],
    scratch_types=[
        pltpu.VMEM((BPW,), jnp.int32),
        pltpu.VMEM((BPW,), jnp.int32),
        pltpu.VMEM((BPW // 2, GW), jnp.float32),
        pltpu.VMEM((BPW // 2, GW), jnp.float32),
        pltpu.SemaphoreType.DMA,
        pltpu.SemaphoreType.DMA,
    ],
)
def _sc_gather2(uid_hbm, iid_hbm, utab_hbm, itab_hbm,
                uout_hbm, iout_hbm,
                uidx_v, iidx_v, urows_v, irows_v, usem, isem):
    wid = lax.axis_index("s") * _NC + lax.axis_index("c")
    base = wid * BPW
    ch = BPW // 2
    pltpu.sync_copy(uid_hbm.at[pl.ds(base, BPW)], uidx_v)
    pltpu.sync_copy(iid_hbm.at[pl.ds(base, BPW)], iidx_v)

    def _shift(i, _):
        sl = pl.ds(i * L, L)
        uidx_v[sl] = lax.shift_right_logical(uidx_v[sl], 2)
        iidx_v[sl] = lax.shift_right_logical(iidx_v[sl], 2)
        return 0

    lax.fori_loop(0, BPW // L, _shift, 0)
    for k in range(2):
        cu = pltpu.async_copy(utab_hbm.at[uidx_v.at[pl.ds(k * ch, ch)]],
                              urows_v, usem)
        ci = pltpu.async_copy(itab_hbm.at[iidx_v.at[pl.ds(k * ch, ch)]],
                              irows_v, isem)
        cu.wait()
        pltpu.sync_copy(urows_v, uout_hbm.at[pl.ds(base + k * ch, ch)])
        ci.wait()
        pltpu.sync_copy(irows_v, iout_hbm.at[pl.ds(base + k * ch, ch)])


BLK = 2048


def _dense_body(uemb4, iemb4, uid_b, iid_b, uf, itf, wuf, buf_, wif, bif_,
                wufc_t, wufc_b, wifc_t, wifc_b, out):
    uc = uid_b[...] & (PACK - 1)
    ic = iid_b[...] & (PACK - 1)
    uemb = jnp.zeros((BLK, E), jnp.float32)
    iemb = jnp.zeros((BLK, E), jnp.float32)
    for c in range(PACK):
        uemb = uemb + jnp.where(uc == c, uemb4[:, c * E:(c + 1) * E], 0.0)
        iemb = iemb + jnp.where(ic == c, iemb4[:, c * E:(c + 1) * E], 0.0)
    ufe = jnp.dot(uf[...], wuf[...], preferred_element_type=jnp.float32) + buf_[...]
    ife = jnp.dot(itf[...], wif[...], preferred_element_type=jnp.float32) + bif_[...]
    fu = jnp.dot(uemb, wufc_t[...], preferred_element_type=jnp.float32)
    fu = fu + jnp.dot(ufe, wufc_b[...], preferred_element_type=jnp.float32)
    fi = jnp.dot(iemb, wifc_t[...], preferred_element_type=jnp.float32)
    fi = fi + jnp.dot(ife, wifc_b[...], preferred_element_type=jnp.float32)
    fu = jnp.maximum(fu, 0.0)
    fi = jnp.maximum(fi, 0.0)
    out[...] = jnp.sum(fu * fi, axis=1, keepdims=True)


def _dense(uemb4, iemb4, uid2, iid2, uf, itf, wuf, buf_, wif, bif_, wufc, wifc):
    grid = (B // BLK,)
    row_spec = lambda w: pl.BlockSpec((BLK, w), lambda i: (i, 0))
    full = lambda a: pl.BlockSpec(a.shape, lambda i: (0,) * a.ndim)
    wufc_t, wufc_b = wufc[:E], wufc[E:]
    wifc_t, wifc_b = wifc[:E], wifc[E:]
    buf2 = buf_.reshape(1, E)
    bif2 = bif_.reshape(1, E)
    return pl.pallas_call(
        _dense_body,
        grid=grid,
        in_specs=[
            row_spec(GW), row_spec(GW), row_spec(1), row_spec(1),
            row_spec(NF), row_spec(NF),
            full(wuf), full(buf2), full(wif), full(bif2),
            full(wufc_t), full(wufc_b), full(wifc_t), full(wifc_b),
        ],
        out_specs=pl.BlockSpec((BLK, 1), lambda i: (i, 0)),
        out_shape=jax.ShapeDtypeStruct((B, 1), jnp.float32),
    )(uemb4, iemb4, uid2, iid2, uf, itf, wuf, buf2, wif, bif2,
      wufc_t, wufc_b, wifc_t, wifc_b)


def kernel(user_id, target_item_id, history_item_id, history_len,
           user_features, item_features, user_table, item_table,
           W_uf, b_uf, W_if, b_if, W_ufc, W_ifc):
    uid = user_id.reshape(B).astype(jnp.int32)
    iid = target_item_id.reshape(B).astype(jnp.int32)
    utab4 = user_table.reshape(-1, GW)
    itab4 = item_table.reshape(-1, GW)
    uemb4, iemb4 = _sc_gather2(uid, iid, utab4, itab4)
    return _dense(uemb4, iemb4,
                  uid.reshape(B, 1), iid.reshape(B, 1),
                  user_features, item_features,
                  W_uf, b_uf, W_if, b_if, W_ufc, W_ifc)


# speedup vs baseline: 1.5313x; 1.5313x over previous
import functools

import jax
import jax.numpy as jnp
from jax import lax
from jax.experimental import pallas as pl
from jax.experimental.pallas import tpu as pltpu
from jax.experimental.pallas import tpu_sc as plsc

B = 16384
E = 32
NF = 64

_info = plsc.get_sparse_core_info()
_NC, _NS = _info.num_cores, _info.num_subcores
NW = _NC * _NS
BPW = B // NW
CH = BPW // 2


@functools.partial(
    pl.kernel,
    mesh=plsc.VectorSubcoreMesh(core_axis_name="c", subcore_axis_name="s"),
    compiler_params=pltpu.CompilerParams(use_tc_tiling_on_sc=True),
    out_type=[
        jax.ShapeDtypeStruct((B, E), jnp.float32),
        jax.ShapeDtypeStruct((B, E), jnp.float32),
    ],
    scratch_types=[
        pltpu.VMEM((BPW,), jnp.int32),
        pltpu.VMEM((BPW,), jnp.int32),
        pltpu.VMEM((CH, E), jnp.float32),
        pltpu.VMEM((CH, E), jnp.float32),
        pltpu.SemaphoreType.DMA,
        pltpu.SemaphoreType.DMA,
    ],
)
def _sc_gather2(uid_hbm, iid_hbm, utab_hbm, itab_hbm,
                uout_hbm, iout_hbm,
                uidx_v, iidx_v, urows_v, irows_v, usem, isem):
    wid = lax.axis_index("s") * _NC + lax.axis_index("c")
    base = wid * BPW
    pltpu.sync_copy(uid_hbm.at[pl.ds(base, BPW)], uidx_v)
    pltpu.sync_copy(iid_hbm.at[pl.ds(base, BPW)], iidx_v)
    for k in range(BPW // CH):
        def _fire(g, _):
            uvec = uidx_v[pl.ds(k * CH + g * 16, 16)]
            ivec = iidx_v[pl.ds(k * CH + g * 16, 16)]
            for l in range(16):
                pltpu.async_copy(utab_hbm.at[pl.ds(uvec[l], 1)],
                                 urows_v.at[pl.ds(g * 16 + l, 1)], usem)
                pltpu.async_copy(itab_hbm.at[pl.ds(ivec[l], 1)],
                                 irows_v.at[pl.ds(g * 16 + l, 1)], isem)
            return 0

        lax.fori_loop(0, CH // 16, _fire, 0)
        pltpu.make_async_copy(utab_hbm.at[pl.ds(0, CH)],
                              urows_v,
                              usem).wait()
        pltpu.sync_copy(urows_v,
                        uout_hbm.at[pl.ds(base + k * CH, CH)])
        pltpu.make_async_copy(itab_hbm.at[pl.ds(0, CH)],
                              irows_v,
                              isem).wait()
        pltpu.sync_copy(irows_v,
                        iout_hbm.at[pl.ds(base + k * CH, CH)])


BLK = 2048


def _dense_body(uemb, iemb, uf, itf, wuf, buf_, wif, bif_,
                wufc_t, wufc_b, wifc_t, wifc_b, out):
    ufe = jnp.dot(uf[...], wuf[...], preferred_element_type=jnp.float32) + buf_[...]
    ife = jnp.dot(itf[...], wif[...], preferred_element_type=jnp.float32) + bif_[...]
    fu = jnp.dot(uemb[...], wufc_t[...], preferred_element_type=jnp.float32)
    fu = fu + jnp.dot(ufe, wufc_b[...], preferred_element_type=jnp.float32)
    fi = jnp.dot(iemb[...], wifc_t[...], preferred_element_type=jnp.float32)
    fi = fi + jnp.dot(ife, wifc_b[...], preferred_element_type=jnp.float32)
    fu = jnp.maximum(fu, 0.0)
    fi = jnp.maximum(fi, 0.0)
    out[...] = jnp.sum(fu * fi, axis=1, keepdims=True)


def _dense(uemb, iemb, uf, itf, wuf, buf_, wif, bif_, wufc, wifc):
    grid = (B // BLK,)
    row_spec = lambda w: pl.BlockSpec((BLK, w), lambda i: (i, 0))
    full = lambda a: pl.BlockSpec(a.shape, lambda i: (0,) * a.ndim)
    wufc_t, wufc_b = wufc[:E], wufc[E:]
    wifc_t, wifc_b = wifc[:E], wifc[E:]
    buf2 = buf_.reshape(1, E)
    bif2 = bif_.reshape(1, E)
    return pl.pallas_call(
        _dense_body,
        grid=grid,
        in_specs=[
            row_spec(E), row_spec(E), row_spec(NF), row_spec(NF),
            full(wuf), full(buf2), full(wif), full(bif2),
            full(wufc_t), full(wufc_b), full(wifc_t), full(wifc_b),
        ],
        out_specs=pl.BlockSpec((BLK, 1), lambda i: (i, 0)),
        out_shape=jax.ShapeDtypeStruct((B, 1), jnp.float32),
    )(uemb, iemb, uf, itf, wuf, buf2, wif, bif2,
      wufc_t, wufc_b, wifc_t, wifc_b)


def kernel(user_id, target_item_id, history_item_id, history_len,
           user_features, item_features, user_table, item_table,
           W_uf, b_uf, W_if, b_if, W_ufc, W_ifc):
    uid = user_id.reshape(B).astype(jnp.int32)
    iid = target_item_id.reshape(B).astype(jnp.int32)
    uemb, iemb = _sc_gather2(uid, iid, user_table, item_table)
    return _dense(uemb, iemb, user_features, item_features,
                  W_uf, b_uf, W_if, b_if, W_ufc, W_ifc)


# speedup vs baseline: 1.6459x; 1.0748x over previous
import functools

import jax
import jax.numpy as jnp
from jax import lax
from jax.experimental import pallas as pl
from jax.experimental.pallas import tpu as pltpu
from jax.experimental.pallas import tpu_sc as plsc

B = 16384
V = 1000000
E = 32
NF = 64
PACK = 4
GW = PACK * E

_info = plsc.get_sparse_core_info()
_NC, _NS = _info.num_cores, _info.num_subcores
NW = _NC * _NS
BPW = B // NW


TV = 8192
_TGRID = -(-V // TV)


def _pack_body(tabT, out):
    x = tabT[...]
    t = jnp.transpose(x)
    q = TV // PACK
    out[...] = jnp.concatenate([t[c * q:(c + 1) * q] for c in range(PACK)],
                               axis=1)


def _pack(tabT):
    return pl.pallas_call(
        _pack_body,
        grid=(_TGRID,),
        in_specs=[pl.BlockSpec((E, TV), lambda i: (0, i))],
        out_specs=pl.BlockSpec((TV // PACK, GW), lambda i: (i, 0)),
        out_shape=jax.ShapeDtypeStruct((_TGRID * TV // PACK, GW), jnp.float32),
    )(tabT)


CH = BPW // 2


@functools.partial(
    pl.kernel,
    mesh=plsc.VectorSubcoreMesh(core_axis_name="c", subcore_axis_name="s"),
    compiler_params=pltpu.CompilerParams(use_tc_tiling_on_sc=True),
    out_type=[
        jax.ShapeDtypeStruct((B, GW), jnp.float32),
        jax.ShapeDtypeStruct((B, GW), jnp.float32),
    ],
    scratch_types=[
        pltpu.VMEM((BPW,), jnp.int32),
        pltpu.VMEM((BPW,), jnp.int32),
        pltpu.VMEM((CH, GW), jnp.float32),
        pltpu.VMEM((CH, GW), jnp.float32),
        pltpu.SemaphoreType.DMA,
        pltpu.SemaphoreType.DMA,
    ],
)
def _sc_gather2(uid_hbm, iid_hbm, utab_hbm, itab_hbm,
                uout_hbm, iout_hbm,
                uidx_v, iidx_v, urows_v, irows_v, usem, isem):
    wid = lax.axis_index("s") * _NC + lax.axis_index("c")
    base = wid * BPW
    pltpu.sync_copy(uid_hbm.at[pl.ds(base, BPW)], uidx_v)
    pltpu.sync_copy(iid_hbm.at[pl.ds(base, BPW)], iidx_v)

    def _row(v):
        return lax.bitwise_or(
            lax.shift_left(lax.shift_right_logical(v, 13), 11),
            lax.bitwise_and(v, 2047))

    def _shift(i, _):
        sl = pl.ds(i * 16, 16)
        uidx_v[sl] = _row(uidx_v[sl])
        iidx_v[sl] = _row(iidx_v[sl])
        return 0

    lax.fori_loop(0, BPW // 16, _shift, 0)
    for k in range(BPW // CH):
        cu = pltpu.async_copy(utab_hbm.at[uidx_v.at[pl.ds(k * CH, CH)]],
                              urows_v, usem)
        ci = pltpu.async_copy(itab_hbm.at[iidx_v.at[pl.ds(k * CH, CH)]],
                              irows_v, isem)
        cu.wait()
        pltpu.sync_copy(urows_v, uout_hbm.at[pl.ds(base + k * CH, CH)])
        ci.wait()
        pltpu.sync_copy(irows_v, iout_hbm.at[pl.ds(base + k * CH, CH)])


BLK = 2048


def _dense_body(uemb4, iemb4, uid_b, iid_b, uf, itf, wuf, buf_, wif, bif_,
                wufc_t, wufc_b, wifc_t, wifc_b, out):
    uc = (uid_b[...] >> 11) & (PACK - 1)
    ic = (iid_b[...] >> 11) & (PACK - 1)
    uemb = jnp.zeros((BLK, E), jnp.float32)
    iemb = jnp.zeros((BLK, E), jnp.float32)
    for c in range(PACK):
        uemb = uemb + jnp.where(uc == c, uemb4[:, c * E:(c + 1) * E], 0.0)
        iemb = iemb + jnp.where(ic == c, iemb4[:, c * E:(c + 1) * E], 0.0)
    ufe = jnp.dot(uf[...], wuf[...], preferred_element_type=jnp.float32) + buf_[...]
    ife = jnp.dot(itf[...], wif[...], preferred_element_type=jnp.float32) + bif_[...]
    fu = jnp.dot(uemb, wufc_t[...], preferred_element_type=jnp.float32)
    fu = fu + jnp.dot(ufe, wufc_b[...], preferred_element_type=jnp.float32)
    fi = jnp.dot(iemb, wifc_t[...], preferred_element_type=jnp.float32)
    fi = fi + jnp.dot(ife, wifc_b[...], preferred_element_type=jnp.float32)
    fu = jnp.maximum(fu, 0.0)
    fi = jnp.maximum(fi, 0.0)
    out[...] = jnp.sum(fu * fi, axis=1, keepdims=True)


def _dense(uemb4, iemb4, uid2, iid2, uf, itf, wuf, buf_, wif, bif_, wufc, wifc):
    grid = (B // BLK,)
    row_spec = lambda w: pl.BlockSpec((BLK, w), lambda i: (i, 0))
    full = lambda a: pl.BlockSpec(a.shape, lambda i: (0,) * a.ndim)
    wufc_t, wufc_b = wufc[:E], wufc[E:]
    wifc_t, wifc_b = wifc[:E], wifc[E:]
    buf2 = buf_.reshape(1, E)
    bif2 = bif_.reshape(1, E)
    return pl.pallas_call(
        _dense_body,
        grid=grid,
        in_specs=[
            row_spec(GW), row_spec(GW), row_spec(1), row_spec(1),
            row_spec(NF), row_spec(NF),
            full(wuf), full(buf2), full(wif), full(bif2),
            full(wufc_t), full(wufc_b), full(wifc_t), full(wifc_b),
        ],
        out_specs=pl.BlockSpec((BLK, 1), lambda i: (i, 0)),
        out_shape=jax.ShapeDtypeStruct((B, 1), jnp.float32),
    )(uemb4, iemb4, uid2, iid2, uf, itf, wuf, buf2, wif, bif2,
      wufc_t, wufc_b, wifc_t, wifc_b)


def kernel(user_id, target_item_id, history_item_id, history_len,
           user_features, item_features, user_table, item_table,
           W_uf, b_uf, W_if, b_if, W_ufc, W_ifc):
    uid = user_id.reshape(B).astype(jnp.int32)
    iid = target_item_id.reshape(B).astype(jnp.int32)
    utab4 = _pack(user_table.T)
    itab4 = _pack(item_table.T)
    uemb4, iemb4 = _sc_gather2(uid, iid, utab4, itab4)
    return _dense(uemb4, iemb4,
                  uid.reshape(B, 1), iid.reshape(B, 1),
                  user_features, item_features,
                  W_uf, b_uf, W_if, b_if, W_ufc, W_ifc)


# speedup vs baseline: 2.8583x; 1.7366x over previous
import functools

import jax
import jax.numpy as jnp
from jax import lax
from jax.experimental import pallas as pl
from jax.experimental.pallas import tpu as pltpu
from jax.experimental.pallas import tpu_sc as plsc

B = 16384
V = 1000000
E = 32
NF = 64
PACK = 4
GW = PACK * E

_info = plsc.get_sparse_core_info()
_NC, _NS = _info.num_cores, _info.num_subcores
NW = _NC * _NS
BPW = B // NW


TV = 8192
_TGRID = -(-V // TV)


def _pack_body(tabT, out):
    q = TV // PACK
    y = jnp.concatenate([tabT[:, c * q:(c + 1) * q] for c in range(PACK)],
                        axis=0)
    r = lax.broadcasted_iota(jnp.int32, (GW, GW), 0)
    col = lax.broadcasted_iota(jnp.int32, (GW, GW), 1)
    ident = jnp.where(r == col, 1.0, 0.0).astype(jnp.float32)
    out[...] = lax.dot_general(y, ident, (((0,), (0,)), ((), ())),
                               preferred_element_type=jnp.float32)


def _pack(tabT):
    return pl.pallas_call(
        _pack_body,
        grid=(_TGRID,),
        in_specs=[pl.BlockSpec((E, TV), lambda i: (0, i))],
        out_specs=pl.BlockSpec((TV // PACK, GW), lambda i: (i, 0)),
        out_shape=jax.ShapeDtypeStruct((_TGRID * TV // PACK, GW), jnp.float32),
    )(tabT)


CH = BPW // 2


@functools.partial(
    pl.kernel,
    mesh=plsc.VectorSubcoreMesh(core_axis_name="c", subcore_axis_name="s"),
    compiler_params=pltpu.CompilerParams(use_tc_tiling_on_sc=True),
    out_type=[
        jax.ShapeDtypeStruct((B, GW), jnp.float32),
        jax.ShapeDtypeStruct((B, GW), jnp.float32),
    ],
    scratch_types=[
        pltpu.VMEM((BPW,), jnp.int32),
        pltpu.VMEM((BPW,), jnp.int32),
        pltpu.VMEM((CH, GW), jnp.float32),
        pltpu.VMEM((CH, GW), jnp.float32),
        pltpu.SemaphoreType.DMA,
        pltpu.SemaphoreType.DMA,
    ],
)
def _sc_gather2(uid_hbm, iid_hbm, utab_hbm, itab_hbm,
                uout_hbm, iout_hbm,
                uidx_v, iidx_v, urows_v, irows_v, usem, isem):
    wid = lax.axis_index("s") * _NC + lax.axis_index("c")
    base = wid * BPW
    pltpu.sync_copy(uid_hbm.at[pl.ds(base, BPW)], uidx_v)
    pltpu.sync_copy(iid_hbm.at[pl.ds(base, BPW)], iidx_v)

    def _row(v):
        return lax.bitwise_or(
            lax.shift_left(lax.shift_right_logical(v, 13), 11),
            lax.bitwise_and(v, 2047))

    def _shift(i, _):
        sl = pl.ds(i * 16, 16)
        uidx_v[sl] = _row(uidx_v[sl])
        iidx_v[sl] = _row(iidx_v[sl])
        return 0

    lax.fori_loop(0, BPW // 16, _shift, 0)
    for k in range(BPW // CH):
        cu = pltpu.async_copy(utab_hbm.at[uidx_v.at[pl.ds(k * CH, CH)]],
                              urows_v, usem)
        ci = pltpu.async_copy(itab_hbm.at[iidx_v.at[pl.ds(k * CH, CH)]],
                              irows_v, isem)
        cu.wait()
        pltpu.sync_copy(urows_v, uout_hbm.at[pl.ds(base + k * CH, CH)])
        ci.wait()
        pltpu.sync_copy(irows_v, iout_hbm.at[pl.ds(base + k * CH, CH)])


BLK = 2048


def _tdot(aT, b):
    return lax.dot_general(aT, b, (((0,), (0,)), ((), ())),
                           preferred_element_type=jnp.float32)


def _dense_body(uemb4, iemb4, uid_b, iid_b, ufT, itfT, wuf, buf_, wif, bif_,
                wufc_t, wufc_b, wifc_t, wifc_b, ones_e, out):
    uc = (uid_b[...] >> 11) & (PACK - 1)
    ic = (iid_b[...] >> 11) & (PACK - 1)
    grp = lax.broadcasted_iota(jnp.int32, (BLK, GW), 1) // E
    uemb_m = jnp.where(uc == grp, uemb4[...], 0.0)
    iemb_m = jnp.where(ic == grp, iemb4[...], 0.0)
    ufe = _tdot(ufT[...], wuf[...]) + buf_[...]
    ife = _tdot(itfT[...], wif[...]) + bif_[...]
    fu = jnp.dot(uemb_m, wufc_t[...], preferred_element_type=jnp.float32)
    fu = fu + jnp.dot(ufe, wufc_b[...], preferred_element_type=jnp.float32)
    fi = jnp.dot(iemb_m, wifc_t[...], preferred_element_type=jnp.float32)
    fi = fi + jnp.dot(ife, wifc_b[...], preferred_element_type=jnp.float32)
    fu = jnp.maximum(fu, 0.0)
    fi = jnp.maximum(fi, 0.0)
    out[...] = jnp.dot(fu * fi, ones_e[...],
                       preferred_element_type=jnp.float32)


def _dense(uemb4, iemb4, uid2, iid2, ufT, itfT, wuf, buf_, wif, bif_,
           wufc, wifc):
    grid = (B // BLK,)
    row_spec = lambda w: pl.BlockSpec((BLK, w), lambda i: (i, 0))
    colT_spec = pl.BlockSpec((NF, BLK), lambda i: (0, i))
    full = lambda a: pl.BlockSpec(a.shape, lambda i: (0,) * a.ndim)
    wufc_t = jnp.concatenate([wufc[:E]] * PACK, axis=0)
    wifc_t = jnp.concatenate([wifc[:E]] * PACK, axis=0)
    wufc_b, wifc_b = wufc[E:], wifc[E:]
    buf2 = buf_.reshape(1, E)
    bif2 = bif_.reshape(1, E)
    ones_e = jnp.ones((E, 1), jnp.float32)
    return pl.pallas_call(
        _dense_body,
        grid=grid,
        in_specs=[
            row_spec(GW), row_spec(GW), row_spec(1), row_spec(1),
            colT_spec, colT_spec,
            full(wuf), full(buf2), full(wif), full(bif2),
            full(wufc_t), full(wufc_b), full(wifc_t), full(wifc_b),
            full(ones_e),
        ],
        out_specs=pl.BlockSpec((BLK, 1), lambda i: (i, 0)),
        out_shape=jax.ShapeDtypeStruct((B, 1), jnp.float32),
    )(uemb4, iemb4, uid2, iid2, ufT, itfT, wuf, buf2, wif, bif2,
      wufc_t, wufc_b, wifc_t, wifc_b, ones_e)


def kernel(user_id, target_item_id, history_item_id, history_len,
           user_features, item_features, user_table, item_table,
           W_uf, b_uf, W_if, b_if, W_ufc, W_ifc):
    uid = user_id.reshape(B).astype(jnp.int32)
    iid = target_item_id.reshape(B).astype(jnp.int32)
    utab4 = _pack(user_table.T)
    itab4 = _pack(item_table.T)
    uemb4, iemb4 = _sc_gather2(uid, iid, utab4, itab4)
    return _dense(uemb4, iemb4,
                  uid.reshape(B, 1), iid.reshape(B, 1),
                  user_features.T, item_features.T,
                  W_uf, b_uf, W_if, b_if, W_ufc, W_ifc)


# speedup vs baseline: 3.7188x; 1.3011x over previous
import functools

import jax
import jax.numpy as jnp
from jax import lax
from jax.experimental import pallas as pl
from jax.experimental.pallas import tpu as pltpu
from jax.experimental.pallas import tpu_sc as plsc

B = 16384
V = 1000000
E = 32
NF = 64
PACK = 4
GW = PACK * E

_info = plsc.get_sparse_core_info()
_NC, _NS = _info.num_cores, _info.num_subcores
NW = _NC * _NS
BPW = B // NW


TV = 16384
_TGRID = -(-V // TV)


def _pack_body(tabT, out):
    q = TV // PACK
    y = jnp.concatenate([tabT[:, c * q:(c + 1) * q] for c in range(PACK)],
                        axis=0)
    r = lax.broadcasted_iota(jnp.int32, (GW, GW), 0)
    col = lax.broadcasted_iota(jnp.int32, (GW, GW), 1)
    ident = jnp.where(r == col, 1.0, 0.0).astype(jnp.float32)
    out[...] = lax.dot_general(y, ident, (((0,), (0,)), ((), ())),
                               preferred_element_type=jnp.float32)


def _pack(tabT):
    return pl.pallas_call(
        _pack_body,
        grid=(_TGRID,),
        in_specs=[pl.BlockSpec((E, TV), lambda i: (0, i))],
        out_specs=pl.BlockSpec((TV // PACK, GW), lambda i: (i, 0)),
        out_shape=jax.ShapeDtypeStruct((_TGRID * TV // PACK, GW), jnp.float32),
    )(tabT)


CH = BPW // 2


@functools.partial(
    pl.kernel,
    mesh=plsc.VectorSubcoreMesh(core_axis_name="c", subcore_axis_name="s"),
    compiler_params=pltpu.CompilerParams(use_tc_tiling_on_sc=True),
    out_type=[
        jax.ShapeDtypeStruct((B, GW), jnp.float32),
        jax.ShapeDtypeStruct((B, GW), jnp.float32),
    ],
    scratch_types=[
        pltpu.VMEM((BPW,), jnp.int32),
        pltpu.VMEM((BPW,), jnp.int32),
        pltpu.VMEM((CH, GW), jnp.float32),
        pltpu.VMEM((CH, GW), jnp.float32),
        pltpu.SemaphoreType.DMA,
        pltpu.SemaphoreType.DMA,
    ],
)
def _sc_gather2(uid_hbm, iid_hbm, utab_hbm, itab_hbm,
                uout_hbm, iout_hbm,
                uidx_v, iidx_v, urows_v, irows_v, usem, isem):
    wid = lax.axis_index("s") * _NC + lax.axis_index("c")
    base = wid * BPW
    pltpu.sync_copy(uid_hbm.at[pl.ds(base, BPW)], uidx_v)
    pltpu.sync_copy(iid_hbm.at[pl.ds(base, BPW)], iidx_v)

    def _row(v):
        return lax.bitwise_or(
            lax.shift_left(lax.shift_right_logical(v, 14), 12),
            lax.bitwise_and(v, 4095))

    def _shift(i, _):
        sl = pl.ds(i * 16, 16)
        uidx_v[sl] = _row(uidx_v[sl])
        iidx_v[sl] = _row(iidx_v[sl])
        return 0

    lax.fori_loop(0, BPW // 16, _shift, 0)
    for k in range(BPW // CH):
        cu = pltpu.async_copy(utab_hbm.at[uidx_v.at[pl.ds(k * CH, CH)]],
                              urows_v, usem)
        ci = pltpu.async_copy(itab_hbm.at[iidx_v.at[pl.ds(k * CH, CH)]],
                              irows_v, isem)
        cu.wait()
        pltpu.sync_copy(urows_v, uout_hbm.at[pl.ds(base + k * CH, CH)])
        ci.wait()
        pltpu.sync_copy(irows_v, iout_hbm.at[pl.ds(base + k * CH, CH)])


BLK = 2048


def _tdot(aT, b):
    return lax.dot_general(aT, b, (((0,), (0,)), ((), ())),
                           preferred_element_type=jnp.float32)


def _dense_body(uemb4, iemb4, uid_b, iid_b, ufT, itfT, wuf, buf_, wif, bif_,
                wufc_t, wufc_b, wifc_t, wifc_b, ones_e, out):
    uc = (uid_b[...] >> 12) & (PACK - 1)
    ic = (iid_b[...] >> 12) & (PACK - 1)
    grp = lax.broadcasted_iota(jnp.int32, (BLK, GW), 1) // E
    uemb_m = jnp.where(uc == grp, uemb4[...], 0.0)
    iemb_m = jnp.where(ic == grp, iemb4[...], 0.0)
    ufe = _tdot(ufT[...], wuf[...]) + buf_[...]
    ife = _tdot(itfT[...], wif[...]) + bif_[...]
    fu = jnp.dot(uemb_m, wufc_t[...], preferred_element_type=jnp.float32)
    fu = fu + jnp.dot(ufe, wufc_b[...], preferred_element_type=jnp.float32)
    fi = jnp.dot(iemb_m, wifc_t[...], preferred_element_type=jnp.float32)
    fi = fi + jnp.dot(ife, wifc_b[...], preferred_element_type=jnp.float32)
    fu = jnp.maximum(fu, 0.0)
    fi = jnp.maximum(fi, 0.0)
    out[...] = jnp.dot(fu * fi, ones_e[...],
                       preferred_element_type=jnp.float32)


def _dense(uemb4, iemb4, uid2, iid2, ufT, itfT, wuf, buf_, wif, bif_,
           wufc, wifc):
    grid = (B // BLK,)
    row_spec = lambda w: pl.BlockSpec((BLK, w), lambda i: (i, 0))
    colT_spec = pl.BlockSpec((NF, BLK), lambda i: (0, i))
    full = lambda a: pl.BlockSpec(a.shape, lambda i: (0,) * a.ndim)
    wufc_t = jnp.concatenate([wufc[:E]] * PACK, axis=0)
    wifc_t = jnp.concatenate([wifc[:E]] * PACK, axis=0)
    wufc_b, wifc_b = wufc[E:], wifc[E:]
    buf2 = buf_.reshape(1, E)
    bif2 = bif_.reshape(1, E)
    ones_e = jnp.ones((E, 1), jnp.float32)
    return pl.pallas_call(
        _dense_body,
        grid=grid,
        in_specs=[
            row_spec(GW), row_spec(GW), row_spec(1), row_spec(1),
            colT_spec, colT_spec,
            full(wuf), full(buf2), full(wif), full(bif2),
            full(wufc_t), full(wufc_b), full(wifc_t), full(wifc_b),
            full(ones_e),
        ],
        out_specs=pl.BlockSpec((BLK, 1), lambda i: (i, 0)),
        out_shape=jax.ShapeDtypeStruct((B, 1), jnp.float32),
    )(uemb4, iemb4, uid2, iid2, ufT, itfT, wuf, buf2, wif, bif2,
      wufc_t, wufc_b, wifc_t, wifc_b, ones_e)


def kernel(user_id, target_item_id, history_item_id, history_len,
           user_features, item_features, user_table, item_table,
           W_uf, b_uf, W_if, b_if, W_ufc, W_ifc):
    uid = user_id.reshape(B).astype(jnp.int32)
    iid = target_item_id.reshape(B).astype(jnp.int32)
    utab4 = _pack(user_table.T)
    itab4 = _pack(item_table.T)
    uemb4, iemb4 = _sc_gather2(uid, iid, utab4, itab4)
    return _dense(uemb4, iemb4,
                  uid.reshape(B, 1), iid.reshape(B, 1),
                  user_features.T, item_features.T,
                  W_uf, b_uf, W_if, b_if, W_ufc, W_ifc)


# speedup vs baseline: 4.1873x; 1.1260x over previous
import functools

import jax
import jax.numpy as jnp
from jax import lax
from jax.experimental import pallas as pl
from jax.experimental.pallas import tpu as pltpu
from jax.experimental.pallas import tpu_sc as plsc

B = 16384
V = 1000000
E = 32
NF = 64
PACK = 4
GW = PACK * E

_info = plsc.get_sparse_core_info()
_NC, _NS = _info.num_cores, _info.num_subcores
NW = _NC * _NS
BPW = B // NW


TV = 32768
TVB = TV.bit_length() - 1
Q = TV // PACK
QB = Q.bit_length() - 1
_TGRID = -(-V // TV)


def _pack_body(tabT, out):
    q = TV // PACK
    y = jnp.concatenate([tabT[:, c * q:(c + 1) * q] for c in range(PACK)],
                        axis=0)
    r = lax.broadcasted_iota(jnp.int32, (GW, GW), 0)
    col = lax.broadcasted_iota(jnp.int32, (GW, GW), 1)
    ident = jnp.where(r == col, 1.0, 0.0).astype(jnp.float32)
    out[...] = lax.dot_general(y, ident, (((0,), (0,)), ((), ())),
                               preferred_element_type=jnp.float32)


def _pack(tabT):
    return pl.pallas_call(
        _pack_body,
        grid=(_TGRID,),
        in_specs=[pl.BlockSpec((E, TV), lambda i: (0, i))],
        out_specs=pl.BlockSpec((TV // PACK, GW), lambda i: (i, 0)),
        out_shape=jax.ShapeDtypeStruct((_TGRID * TV // PACK, GW), jnp.float32),
    )(tabT)


CH = BPW // 2


@functools.partial(
    pl.kernel,
    mesh=plsc.VectorSubcoreMesh(core_axis_name="c", subcore_axis_name="s"),
    compiler_params=pltpu.CompilerParams(use_tc_tiling_on_sc=True),
    out_type=[
        jax.ShapeDtypeStruct((B, GW), jnp.float32),
        jax.ShapeDtypeStruct((B, GW), jnp.float32),
    ],
    scratch_types=[
        pltpu.VMEM((BPW,), jnp.int32),
        pltpu.VMEM((BPW,), jnp.int32),
        pltpu.VMEM((CH, GW), jnp.float32),
        pltpu.VMEM((CH, GW), jnp.float32),
        pltpu.SemaphoreType.DMA,
        pltpu.SemaphoreType.DMA,
    ],
)
def _sc_gather2(uid_hbm, iid_hbm, utab_hbm, itab_hbm,
                uout_hbm, iout_hbm,
                uidx_v, iidx_v, urows_v, irows_v, usem, isem):
    wid = lax.axis_index("s") * _NC + lax.axis_index("c")
    base = wid * BPW
    pltpu.sync_copy(uid_hbm.at[pl.ds(base, BPW)], uidx_v)
    pltpu.sync_copy(iid_hbm.at[pl.ds(base, BPW)], iidx_v)

    def _row(v):
        return lax.bitwise_or(
            lax.shift_left(lax.shift_right_logical(v, TVB), QB),
            lax.bitwise_and(v, Q - 1))

    def _shift(i, _):
        sl = pl.ds(i * 16, 16)
        uidx_v[sl] = _row(uidx_v[sl])
        iidx_v[sl] = _row(iidx_v[sl])
        return 0

    lax.fori_loop(0, BPW // 16, _shift, 0)
    for k in range(BPW // CH):
        cu = pltpu.async_copy(utab_hbm.at[uidx_v.at[pl.ds(k * CH, CH)]],
                              urows_v, usem)
        ci = pltpu.async_copy(itab_hbm.at[iidx_v.at[pl.ds(k * CH, CH)]],
                              irows_v, isem)
        cu.wait()
        pltpu.sync_copy(urows_v, uout_hbm.at[pl.ds(base + k * CH, CH)])
        ci.wait()
        pltpu.sync_copy(irows_v, iout_hbm.at[pl.ds(base + k * CH, CH)])


BLK = 2048


def _tdot(aT, b):
    return lax.dot_general(aT, b, (((0,), (0,)), ((), ())),
                           preferred_element_type=jnp.float32)


def _dense_body(uemb4, iemb4, uid_b, iid_b, ufT, itfT, wuf, buf_, wif, bif_,
                wufc_t, wufc_b, wifc_t, wifc_b, ones_e, out):
    uc = (uid_b[...] >> QB) & (PACK - 1)
    ic = (iid_b[...] >> QB) & (PACK - 1)
    grp = lax.broadcasted_iota(jnp.int32, (BLK, GW), 1) // E
    uemb_m = jnp.where(uc == grp, uemb4[...], 0.0)
    iemb_m = jnp.where(ic == grp, iemb4[...], 0.0)
    ufe = _tdot(ufT[...], wuf[...]) + buf_[...]
    ife = _tdot(itfT[...], wif[...]) + bif_[...]
    fu = jnp.dot(uemb_m, wufc_t[...], preferred_element_type=jnp.float32)
    fu = fu + jnp.dot(ufe, wufc_b[...], preferred_element_type=jnp.float32)
    fi = jnp.dot(iemb_m, wifc_t[...], preferred_element_type=jnp.float32)
    fi = fi + jnp.dot(ife, wifc_b[...], preferred_element_type=jnp.float32)
    fu = jnp.maximum(fu, 0.0)
    fi = jnp.maximum(fi, 0.0)
    out[...] = jnp.dot(fu * fi, ones_e[...],
                       preferred_element_type=jnp.float32)


def _dense(uemb4, iemb4, uid2, iid2, ufT, itfT, wuf, buf_, wif, bif_,
           wufc, wifc):
    grid = (B // BLK,)
    row_spec = lambda w: pl.BlockSpec((BLK, w), lambda i: (i, 0))
    colT_spec = pl.BlockSpec((NF, BLK), lambda i: (0, i))
    full = lambda a: pl.BlockSpec(a.shape, lambda i: (0,) * a.ndim)
    wufc_t = jnp.concatenate([wufc[:E]] * PACK, axis=0)
    wifc_t = jnp.concatenate([wifc[:E]] * PACK, axis=0)
    wufc_b, wifc_b = wufc[E:], wifc[E:]
    buf2 = buf_.reshape(1, E)
    bif2 = bif_.reshape(1, E)
    ones_e = jnp.ones((E, 1), jnp.float32)
    return pl.pallas_call(
        _dense_body,
        grid=grid,
        in_specs=[
            row_spec(GW), row_spec(GW), row_spec(1), row_spec(1),
            colT_spec, colT_spec,
            full(wuf), full(buf2), full(wif), full(bif2),
            full(wufc_t), full(wufc_b), full(wifc_t), full(wifc_b),
            full(ones_e),
        ],
        out_specs=pl.BlockSpec((BLK, 1), lambda i: (i, 0)),
        out_shape=jax.ShapeDtypeStruct((B, 1), jnp.float32),
    )(uemb4, iemb4, uid2, iid2, ufT, itfT, wuf, buf2, wif, bif2,
      wufc_t, wufc_b, wifc_t, wifc_b, ones_e)


def kernel(user_id, target_item_id, history_item_id, history_len,
           user_features, item_features, user_table, item_table,
           W_uf, b_uf, W_if, b_if, W_ufc, W_ifc):
    uid = user_id.reshape(B).astype(jnp.int32)
    iid = target_item_id.reshape(B).astype(jnp.int32)
    utab4 = _pack(user_table.T)
    itab4 = _pack(item_table.T)
    uemb4, iemb4 = _sc_gather2(uid, iid, utab4, itab4)
    return _dense(uemb4, iemb4,
                  uid.reshape(B, 1), iid.reshape(B, 1),
                  user_features.T, item_features.T,
                  W_uf, b_uf, W_if, b_if, W_ufc, W_ifc)


# speedup vs baseline: 4.1913x; 1.0010x over previous
import functools

import jax
import jax.numpy as jnp
from jax import lax
from jax.experimental import pallas as pl
from jax.experimental.pallas import tpu as pltpu
from jax.experimental.pallas import tpu_sc as plsc

B = 16384
V = 1000000
E = 32
NF = 64
PACK = 4
GW = PACK * E

_info = plsc.get_sparse_core_info()
_NC, _NS = _info.num_cores, _info.num_subcores
NW = _NC * _NS
BPW = B // NW


TV = 65536
TVB = TV.bit_length() - 1
Q = TV // PACK
QB = Q.bit_length() - 1
_TGRID = -(-V // TV)


def _pack_body(tabT, out):
    q = TV // PACK
    y = jnp.concatenate([tabT[:, c * q:(c + 1) * q] for c in range(PACK)],
                        axis=0)
    r = lax.broadcasted_iota(jnp.int32, (GW, GW), 0)
    col = lax.broadcasted_iota(jnp.int32, (GW, GW), 1)
    ident = jnp.where(r == col, 1.0, 0.0).astype(jnp.float32)
    out[...] = lax.dot_general(y, ident, (((0,), (0,)), ((), ())),
                               preferred_element_type=jnp.float32)


def _pack(tabT):
    return pl.pallas_call(
        _pack_body,
        grid=(_TGRID,),
        in_specs=[pl.BlockSpec((E, TV), lambda i: (0, i))],
        out_specs=pl.BlockSpec((TV // PACK, GW), lambda i: (i, 0)),
        out_shape=jax.ShapeDtypeStruct((_TGRID * TV // PACK, GW), jnp.float32),
    )(tabT)


CH = BPW // 2


@functools.partial(
    pl.kernel,
    mesh=plsc.VectorSubcoreMesh(core_axis_name="c", subcore_axis_name="s"),
    compiler_params=pltpu.CompilerParams(use_tc_tiling_on_sc=True),
    out_type=[
        jax.ShapeDtypeStruct((B, GW), jnp.float32),
        jax.ShapeDtypeStruct((B, GW), jnp.float32),
    ],
    scratch_types=[
        pltpu.VMEM((BPW,), jnp.int32),
        pltpu.VMEM((BPW,), jnp.int32),
        pltpu.VMEM((CH, GW), jnp.float32),
        pltpu.VMEM((CH, GW), jnp.float32),
        pltpu.SemaphoreType.DMA,
        pltpu.SemaphoreType.DMA,
    ],
)
def _sc_gather2(uid_hbm, iid_hbm, utab_hbm, itab_hbm,
                uout_hbm, iout_hbm,
                uidx_v, iidx_v, urows_v, irows_v, usem, isem):
    wid = lax.axis_index("s") * _NC + lax.axis_index("c")
    base = wid * BPW
    pltpu.sync_copy(uid_hbm.at[pl.ds(base, BPW)], uidx_v)
    pltpu.sync_copy(iid_hbm.at[pl.ds(base, BPW)], iidx_v)

    def _row(v):
        return lax.bitwise_or(
            lax.shift_left(lax.shift_right_logical(v, TVB), QB),
            lax.bitwise_and(v, Q - 1))

    def _shift(i, _):
        sl = pl.ds(i * 16, 16)
        uidx_v[sl] = _row(uidx_v[sl])
        iidx_v[sl] = _row(iidx_v[sl])
        return 0

    lax.fori_loop(0, BPW // 16, _shift, 0)
    for k in range(BPW // CH):
        cu = pltpu.async_copy(utab_hbm.at[uidx_v.at[pl.ds(k * CH, CH)]],
                              urows_v, usem)
        ci = pltpu.async_copy(itab_hbm.at[iidx_v.at[pl.ds(k * CH, CH)]],
                              irows_v, isem)
        cu.wait()
        pltpu.sync_copy(urows_v, uout_hbm.at[pl.ds(base + k * CH, CH)])
        ci.wait()
        pltpu.sync_copy(irows_v, iout_hbm.at[pl.ds(base + k * CH, CH)])


BLK = 2048


def _tdot(aT, b):
    return lax.dot_general(aT, b, (((0,), (0,)), ((), ())),
                           preferred_element_type=jnp.float32)


def _dense_body(uemb4, iemb4, uid_b, iid_b, ufT, itfT, wuf, buf_, wif, bif_,
                wufc_t, wufc_b, wifc_t, wifc_b, ones_e, out):
    uc = (uid_b[...] >> QB) & (PACK - 1)
    ic = (iid_b[...] >> QB) & (PACK - 1)
    grp = lax.broadcasted_iota(jnp.int32, (BLK, GW), 1) // E
    uemb_m = jnp.where(uc == grp, uemb4[...], 0.0)
    iemb_m = jnp.where(ic == grp, iemb4[...], 0.0)
    ufe = _tdot(ufT[...], wuf[...]) + buf_[...]
    ife = _tdot(itfT[...], wif[...]) + bif_[...]
    fu = jnp.dot(uemb_m, wufc_t[...], preferred_element_type=jnp.float32)
    fu = fu + jnp.dot(ufe, wufc_b[...], preferred_element_type=jnp.float32)
    fi = jnp.dot(iemb_m, wifc_t[...], preferred_element_type=jnp.float32)
    fi = fi + jnp.dot(ife, wifc_b[...], preferred_element_type=jnp.float32)
    fu = jnp.maximum(fu, 0.0)
    fi = jnp.maximum(fi, 0.0)
    out[...] = jnp.dot(fu * fi, ones_e[...],
                       preferred_element_type=jnp.float32)


def _dense(uemb4, iemb4, uid2, iid2, ufT, itfT, wuf, buf_, wif, bif_,
           wufc, wifc):
    grid = (B // BLK,)
    row_spec = lambda w: pl.BlockSpec((BLK, w), lambda i: (i, 0))
    colT_spec = pl.BlockSpec((NF, BLK), lambda i: (0, i))
    full = lambda a: pl.BlockSpec(a.shape, lambda i: (0,) * a.ndim)
    wufc_t = jnp.concatenate([wufc[:E]] * PACK, axis=0)
    wifc_t = jnp.concatenate([wifc[:E]] * PACK, axis=0)
    wufc_b, wifc_b = wufc[E:], wifc[E:]
    buf2 = buf_.reshape(1, E)
    bif2 = bif_.reshape(1, E)
    ones_e = jnp.ones((E, 1), jnp.float32)
    return pl.pallas_call(
        _dense_body,
        grid=grid,
        in_specs=[
            row_spec(GW), row_spec(GW), row_spec(1), row_spec(1),
            colT_spec, colT_spec,
            full(wuf), full(buf2), full(wif), full(bif2),
            full(wufc_t), full(wufc_b), full(wifc_t), full(wifc_b),
            full(ones_e),
        ],
        out_specs=pl.BlockSpec((BLK, 1), lambda i: (i, 0)),
        out_shape=jax.ShapeDtypeStruct((B, 1), jnp.float32),
    )(uemb4, iemb4, uid2, iid2, ufT, itfT, wuf, buf2, wif, bif2,
      wufc_t, wufc_b, wifc_t, wifc_b, ones_e)


def kernel(user_id, target_item_id, history_item_id, history_len,
           user_features, item_features, user_table, item_table,
           W_uf, b_uf, W_if, b_if, W_ufc, W_ifc):
    uid = user_id.reshape(B).astype(jnp.int32)
    iid = target_item_id.reshape(B).astype(jnp.int32)
    utab4 = _pack(user_table.T)
    itab4 = _pack(item_table.T)
    uemb4, iemb4 = _sc_gather2(uid, iid, utab4, itab4)
    return _dense(uemb4, iemb4,
                  uid.reshape(B, 1), iid.reshape(B, 1),
                  user_features.T, item_features.T,
                  W_uf, b_uf, W_if, b_if, W_ufc, W_ifc)


# speedup vs baseline: 4.4698x; 1.0664x over previous
import functools

import jax
import jax.numpy as jnp
from jax import lax
from jax.experimental import pallas as pl
from jax.experimental.pallas import tpu as pltpu
from jax.experimental.pallas import tpu_sc as plsc

B = 16384
V = 1000000
E = 32
NF = 64
PACK = 4
GW = PACK * E

_info = plsc.get_sparse_core_info()
_NC, _NS = _info.num_cores, _info.num_subcores
NW = _NC * _NS
BPW = B // NW


TV = 65536
TVB = TV.bit_length() - 1
Q = TV // PACK
QB = Q.bit_length() - 1
_TGRID = -(-V // TV)


def _pack_body(tabT, out):
    q = TV // PACK
    y = jnp.concatenate([tabT[:, c * q:(c + 1) * q] for c in range(PACK)],
                        axis=0)
    r = lax.broadcasted_iota(jnp.int32, (GW, GW), 0)
    col = lax.broadcasted_iota(jnp.int32, (GW, GW), 1)
    ident = jnp.where(r == col, 1.0, 0.0).astype(jnp.float32)
    out[...] = lax.dot_general(y, ident, (((0,), (0,)), ((), ())),
                               preferred_element_type=jnp.float32)


def _pack(tabT):
    return pl.pallas_call(
        _pack_body,
        grid=(_TGRID,),
        in_specs=[pl.BlockSpec((E, TV), lambda i: (0, i))],
        out_specs=pl.BlockSpec((TV // PACK, GW), lambda i: (i, 0)),
        out_shape=jax.ShapeDtypeStruct((_TGRID * TV // PACK, GW), jnp.float32),
    )(tabT)


CH = BPW // 2


@functools.partial(
    pl.kernel,
    mesh=plsc.VectorSubcoreMesh(core_axis_name="c", subcore_axis_name="s"),
    compiler_params=pltpu.CompilerParams(use_tc_tiling_on_sc=True),
    out_type=[
        jax.ShapeDtypeStruct((B, GW), jnp.float32),
        jax.ShapeDtypeStruct((B, GW), jnp.float32),
    ],
    scratch_types=[
        pltpu.VMEM((BPW,), jnp.int32),
        pltpu.VMEM((BPW,), jnp.int32),
        pltpu.VMEM((CH, GW), jnp.float32),
        pltpu.VMEM((CH, GW), jnp.float32),
        pltpu.SemaphoreType.DMA,
        pltpu.SemaphoreType.DMA,
    ],
)
def _sc_gather2(uid_hbm, iid_hbm, utab_hbm, itab_hbm,
                uout_hbm, iout_hbm,
                uidx_v, iidx_v, urows_v, irows_v, usem, isem):
    wid = lax.axis_index("s") * _NC + lax.axis_index("c")
    base = wid * BPW
    pltpu.sync_copy(uid_hbm.at[pl.ds(base, BPW)], uidx_v)
    pltpu.sync_copy(iid_hbm.at[pl.ds(base, BPW)], iidx_v)

    def _row(v):
        return lax.bitwise_or(
            lax.shift_left(lax.shift_right_logical(v, TVB), QB),
            lax.bitwise_and(v, Q - 1))

    def _shift(i, _):
        sl = pl.ds(i * 16, 16)
        uidx_v[sl] = _row(uidx_v[sl])
        iidx_v[sl] = _row(iidx_v[sl])
        return 0

    lax.fori_loop(0, BPW // 16, _shift, 0)
    for k in range(BPW // CH):
        cu = pltpu.async_copy(utab_hbm.at[uidx_v.at[pl.ds(k * CH, CH)]],
                              urows_v, usem)
        ci = pltpu.async_copy(itab_hbm.at[iidx_v.at[pl.ds(k * CH, CH)]],
                              irows_v, isem)
        cu.wait()
        pltpu.sync_copy(urows_v, uout_hbm.at[pl.ds(base + k * CH, CH)])
        ci.wait()
        pltpu.sync_copy(irows_v, iout_hbm.at[pl.ds(base + k * CH, CH)])


BLK = 4096


def _tdot(aT, b):
    return lax.dot_general(aT, b, (((0,), (0,)), ((), ())),
                           preferred_element_type=jnp.float32)


def _dense_body(uemb4, iemb4, uid_r, iid_r, ufT, itfT, wuf, buf_, wif, bif_,
                wufc_t, wufc_b, wifc_t, wifc_b, ones_e, ones_1, out):
    ucf = (((uid_r[...] >> QB) & (PACK - 1))).astype(jnp.float32)
    icf = (((iid_r[...] >> QB) & (PACK - 1))).astype(jnp.float32)
    ucT = lax.dot_general(ucf, ones_1[...], (((0,), (0,)), ((), ())),
                          preferred_element_type=jnp.float32)
    icT = lax.dot_general(icf, ones_1[...], (((0,), (0,)), ((), ())),
                          preferred_element_type=jnp.float32)
    grp = (lax.broadcasted_iota(jnp.int32, (BLK, GW), 1) // E).astype(
        jnp.float32)
    uemb_m = jnp.where(ucT == grp, uemb4[...], 0.0)
    iemb_m = jnp.where(icT == grp, iemb4[...], 0.0)
    ufe = _tdot(ufT[...], wuf[...]) + buf_[...]
    ife = _tdot(itfT[...], wif[...]) + bif_[...]
    fu = jnp.dot(uemb_m, wufc_t[...], preferred_element_type=jnp.float32)
    fu = fu + jnp.dot(ufe, wufc_b[...], preferred_element_type=jnp.float32)
    fi = jnp.dot(iemb_m, wifc_t[...], preferred_element_type=jnp.float32)
    fi = fi + jnp.dot(ife, wifc_b[...], preferred_element_type=jnp.float32)
    fu = jnp.maximum(fu, 0.0)
    fi = jnp.maximum(fi, 0.0)
    out[...] = lax.dot_general(ones_e[...], fu * fi,
                               (((0,), (1,)), ((), ())),
                               preferred_element_type=jnp.float32)


def _dense(uemb4, iemb4, uid2, iid2, ufT, itfT, wuf, buf_, wif, bif_,
           wufc, wifc):
    grid = (B // BLK,)
    row_spec = lambda w: pl.BlockSpec((BLK, w), lambda i: (i, 0))
    colT_spec = pl.BlockSpec((NF, BLK), lambda i: (0, i))
    id_spec = pl.BlockSpec((1, BLK), lambda i: (0, i))
    full = lambda a: pl.BlockSpec(a.shape, lambda i: (0,) * a.ndim)
    wufc_t = jnp.concatenate([wufc[:E]] * PACK, axis=0)
    wifc_t = jnp.concatenate([wifc[:E]] * PACK, axis=0)
    wufc_b, wifc_b = wufc[E:], wifc[E:]
    buf2 = buf_.reshape(1, E)
    bif2 = bif_.reshape(1, E)
    ones_e = jnp.ones((E, 1), jnp.float32)
    ones_1 = jnp.ones((1, 1), jnp.float32)
    return pl.pallas_call(
        _dense_body,
        grid=grid,
        in_specs=[
            row_spec(GW), row_spec(GW), id_spec, id_spec,
            colT_spec, colT_spec,
            full(wuf), full(buf2), full(wif), full(bif2),
            full(wufc_t), full(wufc_b), full(wifc_t), full(wifc_b),
            full(ones_e), full(ones_1),
        ],
        out_specs=pl.BlockSpec((1, BLK), lambda i: (0, i)),
        out_shape=jax.ShapeDtypeStruct((1, B), jnp.float32),
    )(uemb4, iemb4, uid2, iid2, ufT, itfT, wuf, buf2, wif, bif2,
      wufc_t, wufc_b, wifc_t, wifc_b, ones_e, ones_1)


def kernel(user_id, target_item_id, history_item_id, history_len,
           user_features, item_features, user_table, item_table,
           W_uf, b_uf, W_if, b_if, W_ufc, W_ifc):
    uid = user_id.reshape(B).astype(jnp.int32)
    iid = target_item_id.reshape(B).astype(jnp.int32)
    utab4 = _pack(user_table.T)
    itab4 = _pack(item_table.T)
    uemb4, iemb4 = _sc_gather2(uid, iid, utab4, itab4)
    outT = _dense(uemb4, iemb4,
                  user_id.astype(jnp.int32).T, target_item_id.astype(jnp.int32).T,
                  user_features.T, item_features.T,
                  W_uf, b_uf, W_if, b_if, W_ufc, W_ifc)
    return outT.reshape(B, 1)


# speedup vs baseline: 4.5028x; 1.0074x over previous
import functools

import jax
import jax.numpy as jnp
from jax import lax
from jax.experimental import pallas as pl
from jax.experimental.pallas import tpu as pltpu
from jax.experimental.pallas import tpu_sc as plsc

B = 16384
V = 1000000
E = 32
NF = 64
PACK = 4
GW = PACK * E

_info = plsc.get_sparse_core_info()
_NC, _NS = _info.num_cores, _info.num_subcores
NW = _NC * _NS
BPW = B // NW


TV = 65536
TVB = TV.bit_length() - 1
Q = TV // PACK
QB = Q.bit_length() - 1
_TGRID = -(-V // TV)


def _pack_body(tabT, out):
    q = TV // PACK
    y = jnp.concatenate([tabT[:, c * q:(c + 1) * q] for c in range(PACK)],
                        axis=0)
    r = lax.broadcasted_iota(jnp.int32, (GW, GW), 0)
    col = lax.broadcasted_iota(jnp.int32, (GW, GW), 1)
    ident = jnp.where(r == col, 1.0, 0.0).astype(jnp.float32)
    out[...] = lax.dot_general(y, ident, (((0,), (0,)), ((), ())),
                               preferred_element_type=jnp.float32)


def _pack(tabT):
    return pl.pallas_call(
        _pack_body,
        grid=(_TGRID,),
        in_specs=[pl.BlockSpec((E, TV), lambda i: (0, i))],
        out_specs=pl.BlockSpec((TV // PACK, GW), lambda i: (i, 0)),
        out_shape=jax.ShapeDtypeStruct((_TGRID * TV // PACK, GW), jnp.float32),
    )(tabT)


CH = BPW // 2


@functools.partial(
    pl.kernel,
    mesh=plsc.VectorSubcoreMesh(core_axis_name="c", subcore_axis_name="s"),
    compiler_params=pltpu.CompilerParams(use_tc_tiling_on_sc=True),
    out_type=jax.ShapeDtypeStruct((B, GW), jnp.float32),
    scratch_types=[
        pltpu.VMEM((BPW,), jnp.int32),
        pltpu.VMEM((CH, GW), jnp.float32),
        pltpu.VMEM((CH, GW), jnp.float32),
        pltpu.SemaphoreType.DMA,
        pltpu.SemaphoreType.DMA,
    ],
)
def _sc_gather1(id_hbm, tab_hbm, out_hbm, idx_v, rows_a, rows_b, sem_a, sem_b):
    wid = lax.axis_index("s") * _NC + lax.axis_index("c")
    base = wid * BPW
    pltpu.sync_copy(id_hbm.at[pl.ds(base, BPW)], idx_v)

    def _row(v):
        return lax.bitwise_or(
            lax.shift_left(lax.shift_right_logical(v, TVB), QB),
            lax.bitwise_and(v, Q - 1))

    def _shift(i, _):
        sl = pl.ds(i * 16, 16)
        idx_v[sl] = _row(idx_v[sl])
        return 0

    lax.fori_loop(0, BPW // 16, _shift, 0)
    ca = pltpu.async_copy(tab_hbm.at[idx_v.at[pl.ds(0, CH)]], rows_a, sem_a)
    cb = pltpu.async_copy(tab_hbm.at[idx_v.at[pl.ds(CH, CH)]], rows_b, sem_b)
    ca.wait()
    pltpu.sync_copy(rows_a, out_hbm.at[pl.ds(base, CH)])
    cb.wait()
    pltpu.sync_copy(rows_b, out_hbm.at[pl.ds(base + CH, CH)])


BLK = 4096


def _tdot(aT, b):
    return lax.dot_general(aT, b, (((0,), (0,)), ((), ())),
                           preferred_element_type=jnp.float32)


def _dense_body(uemb4, iemb4, uid_r, iid_r, ufT, itfT, wuf, buf_, wif, bif_,
                wufc_t, wufc_b, wifc_t, wifc_b, ones_e, ones_1, out):
    ucf = (((uid_r[...] >> QB) & (PACK - 1))).astype(jnp.float32)
    icf = (((iid_r[...] >> QB) & (PACK - 1))).astype(jnp.float32)
    ucT = lax.dot_general(ucf, ones_1[...], (((0,), (0,)), ((), ())),
                          preferred_element_type=jnp.float32)
    icT = lax.dot_general(icf, ones_1[...], (((0,), (0,)), ((), ())),
                          preferred_element_type=jnp.float32)
    grp = (lax.broadcasted_iota(jnp.int32, (BLK, GW), 1) // E).astype(
        jnp.float32)
    uemb_m = jnp.where(ucT == grp, uemb4[...], 0.0)
    iemb_m = jnp.where(icT == grp, iemb4[...], 0.0)
    ufe = _tdot(ufT[...], wuf[...]) + buf_[...]
    ife = _tdot(itfT[...], wif[...]) + bif_[...]
    fu = jnp.dot(uemb_m, wufc_t[...], preferred_element_type=jnp.float32)
    fu = fu + jnp.dot(ufe, wufc_b[...], preferred_element_type=jnp.float32)
    fi = jnp.dot(iemb_m, wifc_t[...], preferred_element_type=jnp.float32)
    fi = fi + jnp.dot(ife, wifc_b[...], preferred_element_type=jnp.float32)
    fu = jnp.maximum(fu, 0.0)
    fi = jnp.maximum(fi, 0.0)
    out[...] = lax.dot_general(ones_e[...], fu * fi,
                               (((0,), (1,)), ((), ())),
                               preferred_element_type=jnp.float32)


def _dense(uemb4, iemb4, uid2, iid2, ufT, itfT, wuf, buf_, wif, bif_,
           wufc, wifc):
    grid = (B // BLK,)
    row_spec = lambda w: pl.BlockSpec((BLK, w), lambda i: (i, 0))
    colT_spec = pl.BlockSpec((NF, BLK), lambda i: (0, i))
    id_spec = pl.BlockSpec((1, BLK), lambda i: (0, i))
    full = lambda a: pl.BlockSpec(a.shape, lambda i: (0,) * a.ndim)
    wufc_t = jnp.concatenate([wufc[:E]] * PACK, axis=0)
    wifc_t = jnp.concatenate([wifc[:E]] * PACK, axis=0)
    wufc_b, wifc_b = wufc[E:], wifc[E:]
    buf2 = buf_.reshape(1, E)
    bif2 = bif_.reshape(1, E)
    ones_e = jnp.ones((E, 1), jnp.float32)
    ones_1 = jnp.ones((1, 1), jnp.float32)
    return pl.pallas_call(
        _dense_body,
        grid=grid,
        in_specs=[
            row_spec(GW), row_spec(GW), id_spec, id_spec,
            colT_spec, colT_spec,
            full(wuf), full(buf2), full(wif), full(bif2),
            full(wufc_t), full(wufc_b), full(wifc_t), full(wifc_b),
            full(ones_e), full(ones_1),
        ],
        out_specs=pl.BlockSpec((1, BLK), lambda i: (0, i)),
        out_shape=jax.ShapeDtypeStruct((1, B), jnp.float32),
    )(uemb4, iemb4, uid2, iid2, ufT, itfT, wuf, buf2, wif, bif2,
      wufc_t, wufc_b, wifc_t, wifc_b, ones_e, ones_1)


def kernel(user_id, target_item_id, history_item_id, history_len,
           user_features, item_features, user_table, item_table,
           W_uf, b_uf, W_if, b_if, W_ufc, W_ifc):
    uid = user_id.reshape(B).astype(jnp.int32)
    iid = target_item_id.reshape(B).astype(jnp.int32)
    utab4 = _pack(user_table.T)
    itab4 = _pack(item_table.T)
    uemb4 = _sc_gather1(uid, utab4)
    iemb4 = _sc_gather1(iid, itab4)
    outT = _dense(uemb4, iemb4,
                  user_id.astype(jnp.int32).T, target_item_id.astype(jnp.int32).T,
                  user_features.T, item_features.T,
                  W_uf, b_uf, W_if, b_if, W_ufc, W_ifc)
    return outT.reshape(B, 1)
